# baseline (speedup 1.0000x reference)
import functools

import jax
import jax.numpy as jnp
from jax import lax
from jax.experimental import pallas as pl
from jax.experimental.pallas import tpu as pltpu
from jax.experimental.pallas import tpu_sc as plsc

B, T, H, W = 8, 64, 128, 128
LOOKUP_WINDOW = 101
PAD = (LOOKUP_WINDOW - 1) // 2

NFRAMES = B * T
FRAME_WORDS = H * W * 3
NBINS = 512
NLANES = 16
NTILES = 32
FRAMES_PER_TILE = NFRAMES // NTILES
PIX_ITERS = (H * W) // NLANES


def _sc_hist_body(frames_hbm, hist_hbm, buf_a, buf_b, lhist, rhist, sem_a, sem_b):
    cid = lax.axis_index("c")
    sid = lax.axis_index("s")
    wid = sid * 2 + cid
    f0 = wid * FRAMES_PER_TILE

    lanes = lax.iota(jnp.int32, NLANES)
    idx_r = lanes * 3
    idx_g = idx_r + 1
    idx_b = idx_r + 2
    lane_off = lanes * NBINS
    ones = jnp.ones((NLANES,), jnp.float32)
    zeros16 = jnp.zeros((NLANES,), jnp.float32)

    def _zbody(i, _):
        lhist[pl.ds(i * NLANES, NLANES)] = zeros16
        return _

    lax.fori_loop(0, (NLANES * NBINS) // NLANES, _zbody, None)

    bufs = (buf_a, buf_b)
    sems = (sem_a, sem_b)
    copies = [None, None]
    copies[0] = pltpu.async_copy(
        frames_hbm.at[pl.ds(f0 * FRAME_WORDS, FRAME_WORDS)], buf_a, sem_a)

    for f in range(FRAMES_PER_TILE):
        cur = f % 2
        nxt = 1 - cur
        if f + 1 < FRAMES_PER_TILE:
            copies[nxt] = pltpu.async_copy(
                frames_hbm.at[pl.ds((f0 + f + 1) * FRAME_WORDS, FRAME_WORDS)],
                bufs[nxt], sems[nxt])
        copies[cur].wait()
        buf = bufs[cur]

        def _hbody(j, _):
            base = j * (NLANES * 3)
            r = plsc.load_gather(buf, [base + idx_r])
            g = plsc.load_gather(buf, [base + idx_g])
            bl = plsc.load_gather(buf, [base + idx_b])
            ri = r.astype(jnp.int32) >> 5
            gi = g.astype(jnp.int32) >> 5
            bi = bl.astype(jnp.int32) >> 5
            bins = (ri << 6) + (gi << 3) + bi
            plsc.addupdate_scatter(lhist, [lane_off + bins], ones)
            return _

        lax.fori_loop(0, PIX_ITERS, _hbody, None)

        def _rbody(ci, _):
            col = ci * NLANES
            acc = zeros16
            for ln in range(NLANES):
                acc = acc + lhist[pl.ds(ln * NBINS + col, NLANES)]
                lhist[pl.ds(ln * NBINS + col, NLANES)] = zeros16
            rhist[pl.ds(col, NLANES)] = acc
            return _

        lax.fori_loop(0, NBINS // NLANES, _rbody, None)

        pltpu.sync_copy(rhist, hist_hbm.at[pl.ds((f0 + f) * NBINS, NBINS)])


def _sc_histograms(frames_flat):
    mesh = plsc.VectorSubcoreMesh(core_axis_name="c", subcore_axis_name="s")
    k = functools.partial(
        pl.kernel,
        mesh=mesh,
        compiler_params=pltpu.CompilerParams(needs_layout_passes=False),
        out_type=jax.ShapeDtypeStruct((NFRAMES * NBINS,), jnp.float32),
        scratch_types=[
            pltpu.VMEM((FRAME_WORDS,), jnp.float32),
            pltpu.VMEM((FRAME_WORDS,), jnp.float32),
            pltpu.VMEM((NLANES * NBINS,), jnp.float32),
            pltpu.VMEM((NBINS,), jnp.float32),
            pltpu.SemaphoreType.DMA,
            pltpu.SemaphoreType.DMA,
        ],
    )(_sc_hist_body)
    return k(frames_flat)


def _tc_post_body(h_ref, o_ref):
    x = h_ref[0]
    s = jnp.sum(x * x, axis=1, keepdims=True)
    n = jnp.maximum(jnp.sqrt(s), 1e-12)
    xn = x / n
    sim = lax.dot_general(
        xn, xn, (((1,), (1,)), ((), ())),
        precision=lax.Precision.HIGHEST,
        preferred_element_type=jnp.float32)
    p = jnp.concatenate(
        [jnp.zeros((T, PAD), jnp.float32), sim,
         jnp.zeros((T, 256 - T - PAD), jnp.float32)], axis=1)
    t_col = lax.broadcasted_iota(jnp.int32, (T, 1), 0)
    for bit in (1, 2, 4, 8, 16, 32):
        rolled = jnp.concatenate([p[:, bit:], p[:, :bit]], axis=1)
        p = jnp.where((t_col & bit) != 0, rolled, p)
    o_ref[0] = p[:, :LOOKUP_WINDOW]


def _tc_post(hist):
    return pl.pallas_call(
        _tc_post_body,
        grid=(B,),
        in_specs=[pl.BlockSpec((1, T, NBINS), lambda b: (b, 0, 0))],
        out_specs=pl.BlockSpec((1, T, LOOKUP_WINDOW), lambda b: (b, 0, 0)),
        out_shape=jax.ShapeDtypeStruct((B, T, LOOKUP_WINDOW), jnp.float32),
    )(hist)


def kernel(inputs):
    frames_flat = inputs.reshape(NFRAMES * FRAME_WORDS)
    hist = _sc_histograms(frames_flat).reshape(B, T, NBINS)
    return _tc_post(hist)


# speedup vs baseline: 42.5011x; 42.5011x over previous
import functools

import jax
import jax.numpy as jnp
from jax import lax
from jax.experimental import pallas as pl
from jax.experimental.pallas import tpu as pltpu
from jax.experimental.pallas import tpu_sc as plsc

B, T, H, W = 8, 64, 128, 128
LOOKUP_WINDOW = 101
PAD = (LOOKUP_WINDOW - 1) // 2

NFRAMES = B * T
FRAME_WORDS = H * W * 3
NBINS = 512
NLANES = 16
NTILES = 32
FRAMES_PER_TILE = NFRAMES // NTILES
PIX_ITERS = (H * W) // NLANES


def _sc_hist_body(frames_hbm, hist_hbm, buf_a, buf_b, lhist, rhist, sem_a, sem_b):
    cid = lax.axis_index("c")
    sid = lax.axis_index("s")
    wid = sid * 2 + cid
    f0 = wid * FRAMES_PER_TILE

    lanes = lax.iota(jnp.int32, NLANES)
    lane_off = lanes * NBINS
    ones = jnp.ones((NLANES,), jnp.float32)
    zeros16 = jnp.zeros((NLANES,), jnp.float32)

    def _zbody(i, _):
        lhist[pl.ds(i * NLANES, NLANES)] = zeros16
        return _

    lax.fori_loop(0, (NLANES * NBINS) // NLANES, _zbody, None)

    bufs = (buf_a, buf_b)
    sems = (sem_a, sem_b)
    copies = [None, None]
    copies[0] = pltpu.async_copy(
        frames_hbm.at[pl.ds(f0 * 3, 3)], buf_a, sem_a)

    for f in range(FRAMES_PER_TILE):
        cur = f % 2
        nxt = 1 - cur
        if f + 1 < FRAMES_PER_TILE:
            copies[nxt] = pltpu.async_copy(
                frames_hbm.at[pl.ds((f0 + f + 1) * 3, 3)],
                bufs[nxt], sems[nxt])
        copies[cur].wait()
        buf = bufs[cur]

        def _hbody(j, _):
            h = j >> 3
            w0 = (j & 7) * NLANES
            r = buf[0, h, pl.ds(w0, NLANES)]
            g = buf[1, h, pl.ds(w0, NLANES)]
            bl = buf[2, h, pl.ds(w0, NLANES)]
            ri = r.astype(jnp.int32) >> 5
            gi = g.astype(jnp.int32) >> 5
            bi = bl.astype(jnp.int32) >> 5
            bins = (ri << 6) + (gi << 3) + bi
            plsc.addupdate_scatter(lhist, [lane_off + bins], ones)
            return _

        lax.fori_loop(0, PIX_ITERS, _hbody, None)

        def _rbody(ci, _):
            col = ci * NLANES
            acc = zeros16
            for ln in range(NLANES):
                acc = acc + lhist[pl.ds(ln * NBINS + col, NLANES)]
                lhist[pl.ds(ln * NBINS + col, NLANES)] = zeros16
            rhist[pl.ds(col, NLANES)] = acc
            return _

        lax.fori_loop(0, NBINS // NLANES, _rbody, None)

        pltpu.sync_copy(rhist, hist_hbm.at[pl.ds((f0 + f) * NBINS, NBINS)])


def _sc_histograms(planes):
    mesh = plsc.VectorSubcoreMesh(core_axis_name="c", subcore_axis_name="s")
    k = functools.partial(
        pl.kernel,
        mesh=mesh,
        compiler_params=pltpu.CompilerParams(needs_layout_passes=False),
        out_type=jax.ShapeDtypeStruct((NFRAMES * NBINS,), jnp.float32),
        scratch_types=[
            pltpu.VMEM((3, H, W), jnp.float32),
            pltpu.VMEM((3, H, W), jnp.float32),
            pltpu.VMEM((NLANES * NBINS,), jnp.float32),
            pltpu.VMEM((NBINS,), jnp.float32),
            pltpu.SemaphoreType.DMA,
            pltpu.SemaphoreType.DMA,
        ],
    )(_sc_hist_body)
    return k(planes)


def _tc_post_body(h_ref, o_ref):
    x = h_ref[...].reshape(T, NBINS)
    s = jnp.sum(x * x, axis=1, keepdims=True)
    n = jnp.maximum(jnp.sqrt(s), 1e-12)
    xn = x / n
    sim = lax.dot_general(
        xn, xn, (((1,), (1,)), ((), ())),
        precision=lax.Precision.HIGHEST,
        preferred_element_type=jnp.float32)
    p = jnp.concatenate(
        [jnp.zeros((T, PAD), jnp.float32), sim,
         jnp.zeros((T, 256 - T - PAD), jnp.float32)], axis=1)
    t_col = lax.broadcasted_iota(jnp.int32, (T, 1), 0)
    for bit in (1, 2, 4, 8, 16, 32):
        rolled = jnp.concatenate([p[:, bit:], p[:, :bit]], axis=1)
        p = jnp.where((t_col & bit) != 0, rolled, p)
    o_ref[0] = p[:, :LOOKUP_WINDOW]


def _tc_post(hist_flat):
    return pl.pallas_call(
        _tc_post_body,
        grid=(B,),
        in_specs=[pl.BlockSpec((T * NBINS,), lambda b: (b,))],
        out_specs=pl.BlockSpec((1, T, LOOKUP_WINDOW), lambda b: (b, 0, 0)),
        out_shape=jax.ShapeDtypeStruct((B, T, LOOKUP_WINDOW), jnp.float32),
    )(hist_flat)


def kernel(inputs):
    planes = jnp.transpose(inputs, (0, 1, 4, 2, 3)).reshape(NFRAMES * 3, H, W)
    hist_flat = _sc_histograms(planes)
    return _tc_post(hist_flat)


# speedup vs baseline: 96.1391x; 2.2620x over previous
import functools

import jax
import jax.numpy as jnp
from jax import lax
from jax.experimental import pallas as pl
from jax.experimental.pallas import tpu as pltpu
from jax.experimental.pallas import tpu_sc as plsc

B, T, H, W = 8, 64, 128, 128
LOOKUP_WINDOW = 101
PAD = (LOOKUP_WINDOW - 1) // 2

NFRAMES = B * T
FRAME_WORDS = H * W * 3
NBINS = 512
NLANES = 16
NTILES = 32
FRAMES_PER_TILE = NFRAMES // NTILES
PIX_ITERS = (H * W) // NLANES


def _sc_hist_body(frames_hbm, hist_hbm, buf_a, buf_b, lhist, rhist, sem_a, sem_b):
    cid = lax.axis_index("c")
    sid = lax.axis_index("s")
    wid = sid * 2 + cid
    f0 = wid * FRAMES_PER_TILE

    lanes = lax.iota(jnp.int32, NLANES)
    lane_off = lanes * NBINS
    ones = jnp.ones((NLANES,), jnp.float32)
    zeros16 = jnp.zeros((NLANES,), jnp.float32)

    @plsc.parallel_loop(0, (NLANES * NBINS) // NLANES, unroll=8)
    def _zbody(i):
        lhist[pl.ds(i * NLANES, NLANES)] = zeros16

    bufs = (buf_a, buf_b)
    sems = (sem_a, sem_b)
    copies = [None, None]
    copies[0] = pltpu.async_copy(
        frames_hbm.at[pl.ds(f0 * 3, 3)], buf_a, sem_a)

    for f in range(FRAMES_PER_TILE):
        cur = f % 2
        nxt = 1 - cur
        if f + 1 < FRAMES_PER_TILE:
            copies[nxt] = pltpu.async_copy(
                frames_hbm.at[pl.ds((f0 + f + 1) * 3, 3)],
                bufs[nxt], sems[nxt])
        copies[cur].wait()
        buf = bufs[cur]

        @plsc.parallel_loop(0, H, unroll=2)
        def _hbody(h):
            for wc in range(W // NLANES):
                w0 = wc * NLANES
                r = buf[0, h, pl.ds(w0, NLANES)]
                g = buf[1, h, pl.ds(w0, NLANES)]
                bl = buf[2, h, pl.ds(w0, NLANES)]
                ri = r.astype(jnp.int32) >> 5
                gi = g.astype(jnp.int32) >> 5
                bi = bl.astype(jnp.int32) >> 5
                bins = (ri << 6) + (gi << 3) + bi
                plsc.addupdate_scatter(lhist, [lane_off + bins], ones)

        @plsc.parallel_loop(0, NBINS // NLANES, unroll=2)
        def _rbody(ci):
            col = ci * NLANES
            acc = zeros16
            for ln in range(NLANES):
                acc = acc + lhist[pl.ds(ln * NBINS + col, NLANES)]
                lhist[pl.ds(ln * NBINS + col, NLANES)] = zeros16
            rhist[pl.ds(col, NLANES)] = acc

        pltpu.sync_copy(rhist, hist_hbm.at[pl.ds((f0 + f) * NBINS, NBINS)])


def _sc_histograms(planes):
    mesh = plsc.VectorSubcoreMesh(core_axis_name="c", subcore_axis_name="s")
    k = functools.partial(
        pl.kernel,
        mesh=mesh,
        compiler_params=pltpu.CompilerParams(needs_layout_passes=False),
        out_type=jax.ShapeDtypeStruct((NFRAMES * NBINS,), jnp.float32),
        scratch_types=[
            pltpu.VMEM((3, H, W), jnp.float32),
            pltpu.VMEM((3, H, W), jnp.float32),
            pltpu.VMEM((NLANES * NBINS,), jnp.float32),
            pltpu.VMEM((NBINS,), jnp.float32),
            pltpu.SemaphoreType.DMA,
            pltpu.SemaphoreType.DMA,
        ],
    )(_sc_hist_body)
    return k(planes)


def _tc_post_body(h_ref, o_ref):
    x = h_ref[...].reshape(T, NBINS)
    s = jnp.sum(x * x, axis=1, keepdims=True)
    n = jnp.maximum(jnp.sqrt(s), 1e-12)
    xn = x / n
    sim = lax.dot_general(
        xn, xn, (((1,), (1,)), ((), ())),
        precision=lax.Precision.HIGHEST,
        preferred_element_type=jnp.float32)
    p = jnp.concatenate(
        [jnp.zeros((T, PAD), jnp.float32), sim,
         jnp.zeros((T, 256 - T - PAD), jnp.float32)], axis=1)
    t_col = lax.broadcasted_iota(jnp.int32, (T, 1), 0)
    for bit in (1, 2, 4, 8, 16, 32):
        rolled = jnp.concatenate([p[:, bit:], p[:, :bit]], axis=1)
        p = jnp.where((t_col & bit) != 0, rolled, p)
    o_ref[0] = p[:, :LOOKUP_WINDOW]


def _tc_post(hist_flat):
    return pl.pallas_call(
        _tc_post_body,
        grid=(B,),
        in_specs=[pl.BlockSpec((T * NBINS,), lambda b: (b,))],
        out_specs=pl.BlockSpec((1, T, LOOKUP_WINDOW), lambda b: (b, 0, 0)),
        out_shape=jax.ShapeDtypeStruct((B, T, LOOKUP_WINDOW), jnp.float32),
    )(hist_flat)


def kernel(inputs):
    planes = jnp.transpose(inputs, (0, 1, 4, 2, 3)).reshape(NFRAMES * 3, H, W)
    hist_flat = _sc_histograms(planes)
    return _tc_post(hist_flat)


# speedup vs baseline: 104.3568x; 1.0855x over previous
import functools

import jax
import jax.numpy as jnp
from jax import lax
from jax.experimental import pallas as pl
from jax.experimental.pallas import tpu as pltpu
from jax.experimental.pallas import tpu_sc as plsc

B, T, H, W = 8, 64, 128, 128
LOOKUP_WINDOW = 101
PAD = (LOOKUP_WINDOW - 1) // 2

NFRAMES = B * T
FRAME_WORDS = H * W * 3
NBINS = 512
NLANES = 16
NTILES = 32
FRAMES_PER_TILE = NFRAMES // NTILES
PIX_ITERS = (H * W) // NLANES


def _sc_hist_body(frames_hbm, hist_hbm, buf_a, buf_b, lhist, rhist, sem_a, sem_b):
    cid = lax.axis_index("c")
    sid = lax.axis_index("s")
    wid = sid * 2 + cid
    f0 = wid * FRAMES_PER_TILE

    lanes = lax.iota(jnp.int32, NLANES)
    lane_off = lanes * NBINS
    ones = jnp.ones((NLANES,), jnp.float32)
    zeros16 = jnp.zeros((NLANES,), jnp.float32)

    @plsc.parallel_loop(0, (NLANES * NBINS) // NLANES, unroll=8)
    def _zbody(i):
        lhist[pl.ds(i * NLANES, NLANES)] = zeros16

    def _process(buf, fid):
        @plsc.parallel_loop(0, H, unroll=4)
        def _hbody(h):
            for wc in range(W // NLANES):
                w0 = wc * NLANES
                r = buf[0, h, pl.ds(w0, NLANES)]
                g = buf[1, h, pl.ds(w0, NLANES)]
                bl = buf[2, h, pl.ds(w0, NLANES)]
                ri = r.astype(jnp.int32) >> 5
                gi = g.astype(jnp.int32) >> 5
                bi = bl.astype(jnp.int32) >> 5
                bins = (ri << 6) + (gi << 3) + bi
                plsc.addupdate_scatter(lhist, [lane_off + bins], ones)

        @plsc.parallel_loop(0, NBINS // NLANES, unroll=2)
        def _rbody(ci):
            col = ci * NLANES
            acc = zeros16
            for ln in range(NLANES):
                acc = acc + lhist[pl.ds(ln * NBINS + col, NLANES)]
                lhist[pl.ds(ln * NBINS + col, NLANES)] = zeros16
            rhist[pl.ds(col, NLANES)] = acc

        pltpu.sync_copy(rhist, hist_hbm.at[pl.ds(fid * NBINS, NBINS)])

    pltpu.async_copy(frames_hbm.at[pl.ds(f0 * 3, 3)], buf_a, sem_a)
    pltpu.async_copy(frames_hbm.at[pl.ds((f0 + 1) * 3, 3)], buf_b, sem_b)

    def _pair(p, _):
        f_a = f0 + 2 * p

        pltpu.make_async_copy(
            frames_hbm.at[pl.ds(f_a * 3, 3)], buf_a, sem_a).wait()
        _process(buf_a, f_a)

        @pl.when(p < FRAMES_PER_TILE // 2 - 1)
        def _():
            pltpu.async_copy(
                frames_hbm.at[pl.ds((f_a + 2) * 3, 3)], buf_a, sem_a)

        pltpu.make_async_copy(
            frames_hbm.at[pl.ds((f_a + 1) * 3, 3)], buf_b, sem_b).wait()
        _process(buf_b, f_a + 1)

        @pl.when(p < FRAMES_PER_TILE // 2 - 1)
        def _():
            pltpu.async_copy(
                frames_hbm.at[pl.ds((f_a + 3) * 3, 3)], buf_b, sem_b)

        return _

    lax.fori_loop(0, FRAMES_PER_TILE // 2, _pair, None)


def _sc_histograms(planes):
    mesh = plsc.VectorSubcoreMesh(core_axis_name="c", subcore_axis_name="s")
    k = functools.partial(
        pl.kernel,
        mesh=mesh,
        compiler_params=pltpu.CompilerParams(needs_layout_passes=False),
        out_type=jax.ShapeDtypeStruct((NFRAMES * NBINS,), jnp.float32),
        scratch_types=[
            pltpu.VMEM((3, H, W), jnp.float32),
            pltpu.VMEM((3, H, W), jnp.float32),
            pltpu.VMEM((NLANES * NBINS,), jnp.float32),
            pltpu.VMEM((NBINS,), jnp.float32),
            pltpu.SemaphoreType.DMA,
            pltpu.SemaphoreType.DMA,
        ],
    )(_sc_hist_body)
    return k(planes)


def _tc_post_body(h_ref, o_ref):
    x = h_ref[...].reshape(T, NBINS)
    s = jnp.sum(x * x, axis=1, keepdims=True)
    n = jnp.maximum(jnp.sqrt(s), 1e-12)
    xn = x / n
    sim = lax.dot_general(
        xn, xn, (((1,), (1,)), ((), ())),
        precision=lax.Precision.HIGHEST,
        preferred_element_type=jnp.float32)
    p = jnp.concatenate(
        [jnp.zeros((T, PAD), jnp.float32), sim,
         jnp.zeros((T, 256 - T - PAD), jnp.float32)], axis=1)
    t_col = lax.broadcasted_iota(jnp.int32, (T, 1), 0)
    for bit in (1, 2, 4, 8, 16, 32):
        rolled = jnp.concatenate([p[:, bit:], p[:, :bit]], axis=1)
        p = jnp.where((t_col & bit) != 0, rolled, p)
    o_ref[0] = p[:, :LOOKUP_WINDOW]


def _tc_post(hist_flat):
    return pl.pallas_call(
        _tc_post_body,
        grid=(B,),
        in_specs=[pl.BlockSpec((T * NBINS,), lambda b: (b,))],
        out_specs=pl.BlockSpec((1, T, LOOKUP_WINDOW), lambda b: (b, 0, 0)),
        out_shape=jax.ShapeDtypeStruct((B, T, LOOKUP_WINDOW), jnp.float32),
    )(hist_flat)


def kernel(inputs):
    planes = jnp.transpose(inputs, (0, 1, 4, 2, 3)).reshape(NFRAMES * 3, H, W)
    hist_flat = _sc_histograms(planes)
    return _tc_post(hist_flat)
